# Initial kernel scaffold; baseline (speedup 1.0000x reference)
import jax
import jax.numpy as jnp
from jax.experimental import pallas as pl


def kernel(meta_int, meta_float, seq_int, seq_float, W_locS, dw_table, ts_table, lp_table):
    raise NotImplementedError("write your pallas kernel here")


# speedup vs baseline: 1.7714x; 1.7714x over previous
import functools

import jax
import jax.numpy as jnp
from jax import lax
from jax.experimental import pallas as pl
from jax.experimental.pallas import tpu as pltpu
from jax.experimental.pallas import tpu_sc as plsc

_B = 4096
_L = 200
_NPOI = 100
_NW = 32
_BW = _B // _NW
_LANES = 16
_NG = _BW // _LANES


def _sdep_body(mf_ref, mi_ref, w12_ref, dw12_ref, ts12_ref, out_ref):
    mf = mf_ref[...]
    mi = mi_ref[...]
    locs = jnp.dot(mf, w12_ref[...], preferred_element_type=jnp.float32,
                   precision=lax.Precision.HIGHEST)
    dw_oh = (mi[:, 0:1] == lax.broadcasted_iota(jnp.int32, (1, 7), 1)
             ).astype(jnp.float32)
    ts_oh = (mi[:, 1:2] == lax.broadcasted_iota(jnp.int32, (1, 48), 1)
             ).astype(jnp.float32)
    out_ref[...] = (locs
                    + jnp.dot(dw_oh, dw12_ref[...],
                              preferred_element_type=jnp.float32,
                              precision=lax.Precision.HIGHEST)
                    + jnp.dot(ts_oh, ts12_ref[...],
                              preferred_element_type=jnp.float32,
                              precision=lax.Precision.HIGHEST))


def _splat(v):
    return jnp.full((_LANES,), v, jnp.int32)


def _sc_build():
    mesh = plsc.VectorSubcoreMesh(core_axis_name="c", subcore_axis_name="s")
    scratch = [
        pltpu.VMEM((_BW, 12), jnp.float32),
        pltpu.VMEM((_NPOI + 1, 3), jnp.float32),
        pltpu.VMEM((_BW,), jnp.int32),
        pltpu.VMEM((_BW, 5), jnp.float32),
        pltpu.VMEM((_BW, 20), jnp.float32),
        pltpu.VMEM((_BW, 3), jnp.float32),
    ]

    @functools.partial(
        pl.kernel, mesh=mesh,
        out_type=(jax.ShapeDtypeStruct((_L, _B, 20), jnp.float32),
                  jax.ShapeDtypeStruct((_L, _B, 3), jnp.float32)),
        scratch_types=scratch,
        compiler_params=pltpu.CompilerParams(needs_layout_passes=False))
    def sc(sdep_hbm, si_hbm, sf_hbm, lp_hbm, seq_hbm, score_hbm,
           sdep_v, lp_v, si_v, sf_v, out_v, sc_v):
        wid = lax.axis_index("s") * 2 + lax.axis_index("c")
        b0 = wid * _BW
        pltpu.sync_copy(lp_hbm, lp_v)
        pltpu.sync_copy(sdep_hbm.at[pl.ds(b0, _BW), :], sdep_v)
        iota = lax.iota(jnp.int32, _LANES)
        for g in range(_NG):
            rowv = iota + (g * _LANES)
            for j in range(12):
                v = plsc.load_gather(sdep_v, [rowv, _splat(j)])
                plsc.store_scatter(out_v, [rowv, _splat(j)], v)

        def body(l, carry):
            pltpu.sync_copy(si_hbm.at[l, pl.ds(b0, _BW)], si_v)
            pltpu.sync_copy(sf_hbm.at[l, pl.ds(b0, _BW), :], sf_v)
            for g in range(_NG):
                rowv = iota + (g * _LANES)
                siv = si_v[pl.ds(g * _LANES, _LANES)] + 1
                for j in range(3):
                    lv = plsc.load_gather(lp_v, [siv, _splat(j)])
                    plsc.store_scatter(out_v, [rowv, _splat(12 + j)], lv)
                vs = []
                for j in range(5):
                    vj = plsc.load_gather(sf_v, [rowv, _splat(j)])
                    vs.append(vj)
                    plsc.store_scatter(out_v, [rowv, _splat(15 + j)], vj)
                speed = 1.0 / (vs[2] + 1.0)
                plsc.store_scatter(sc_v, [rowv, _splat(0)], speed)
                plsc.store_scatter(sc_v, [rowv, _splat(1)], vs[3])
                plsc.store_scatter(sc_v, [rowv, _splat(2)], vs[4])
            pltpu.sync_copy(out_v, seq_hbm.at[l, pl.ds(b0, _BW), :])
            pltpu.sync_copy(sc_v, score_hbm.at[l, pl.ds(b0, _BW), :])
            return carry

        lax.fori_loop(0, _L, body, 0)

    return sc


_sc_kernel = _sc_build()


def kernel(meta_int, meta_float, seq_int, seq_float, W_locS, dw_table,
           ts_table, lp_table):
    mi = meta_int.astype(jnp.int32)
    mf = jnp.squeeze(meta_float, -1)
    si = jnp.squeeze(seq_int, -1).astype(jnp.int32)
    w12 = jnp.zeros((_NPOI, 12), jnp.float32).at[:, 0:3].set(W_locS.T)
    dw12 = jnp.zeros((7, 12), jnp.float32).at[:, 3:6].set(dw_table)
    ts12 = jnp.zeros((48, 12), jnp.float32).at[:, 6:12].set(ts_table)
    s_depart = pl.pallas_call(
        _sdep_body,
        out_shape=jax.ShapeDtypeStruct((_B, 12), jnp.float32),
    )(mf, mi, w12, dw12, ts12)
    seq, score = _sc_kernel(s_depart, si, seq_float, lp_table)
    return (s_depart, seq, score)


# speedup vs baseline: 2.4706x; 1.3947x over previous
import functools

import jax
import jax.numpy as jnp
from jax import lax
from jax.experimental import pallas as pl
from jax.experimental.pallas import tpu as pltpu
from jax.experimental.pallas import tpu_sc as plsc

_B = 4096
_L = 200
_NPOI = 100
_NW = 32
_BW = _B // _NW
_LANES = 16
_NG = _BW // _LANES


def _sdep_body(mf_ref, mi_ref, w12_ref, dw12_ref, ts12_ref, out_ref):
    mf = mf_ref[...]
    mi = mi_ref[...]
    locs = jnp.dot(mf, w12_ref[...], preferred_element_type=jnp.float32,
                   precision=lax.Precision.HIGHEST)
    dw_oh = (mi[:, 0:1] == lax.broadcasted_iota(jnp.int32, (1, 7), 1)
             ).astype(jnp.float32)
    ts_oh = (mi[:, 1:2] == lax.broadcasted_iota(jnp.int32, (1, 48), 1)
             ).astype(jnp.float32)
    out_ref[...] = (locs
                    + jnp.dot(dw_oh, dw12_ref[...],
                              preferred_element_type=jnp.float32,
                              precision=lax.Precision.HIGHEST)
                    + jnp.dot(ts_oh, ts12_ref[...],
                              preferred_element_type=jnp.float32,
                              precision=lax.Precision.HIGHEST))


def _splat(v):
    return jnp.full((_LANES,), v, jnp.int32)


def _sc_build():
    mesh = plsc.VectorSubcoreMesh(core_axis_name="c", subcore_axis_name="s")
    scratch = [
        pltpu.VMEM((_BW, 12), jnp.float32),
        pltpu.VMEM((_NPOI + 1, 3), jnp.float32),
        pltpu.VMEM((_BW,), jnp.int32),
        pltpu.VMEM((_BW,), jnp.int32),
        pltpu.VMEM((_BW, 5), jnp.float32),
        pltpu.VMEM((_BW, 5), jnp.float32),
        pltpu.VMEM((_BW, 20), jnp.float32),
        pltpu.VMEM((_BW, 20), jnp.float32),
        pltpu.VMEM((_BW, 3), jnp.float32),
        pltpu.VMEM((_BW, 3), jnp.float32),
        pltpu.SemaphoreType.DMA,
        pltpu.SemaphoreType.DMA,
        pltpu.SemaphoreType.DMA,
        pltpu.SemaphoreType.DMA,
    ]

    @functools.partial(
        pl.kernel, mesh=mesh,
        out_type=(jax.ShapeDtypeStruct((_L, _B, 20), jnp.float32),
                  jax.ShapeDtypeStruct((_L, _B, 3), jnp.float32)),
        scratch_types=scratch,
        compiler_params=pltpu.CompilerParams(needs_layout_passes=False))
    def sc(sdep_hbm, si_hbm, sf_hbm, lp_hbm, seq_hbm, score_hbm,
           sdep_v, lp_v, si0, si1, sf0, sf1, out0, out1, sc0, sc1,
           isem0, isem1, osem0, osem1):
        wid = lax.axis_index("s") * 2 + lax.axis_index("c")
        b0 = wid * _BW
        bufs = ((si0, sf0, out0, sc0, isem0, osem0),
                (si1, sf1, out1, sc1, isem1, osem1))
        pltpu.sync_copy(lp_hbm, lp_v)
        pltpu.sync_copy(sdep_hbm.at[pl.ds(b0, _BW), :], sdep_v)
        iota = lax.iota(jnp.int32, _LANES)
        for g in range(_NG):
            rowv = iota + (g * _LANES)
            for j in range(12):
                v = plsc.load_gather(sdep_v, [rowv, _splat(j)])
                plsc.store_scatter(out0, [rowv, _splat(j)], v)
                plsc.store_scatter(out1, [rowv, _splat(j)], v)

        def start_in(l, si_v, sf_v, isem):
            pltpu.async_copy(si_hbm.at[l, pl.ds(b0, _BW)], si_v, isem)
            pltpu.async_copy(sf_hbm.at[l, pl.ds(b0, _BW), :], sf_v, isem)

        def wait_in(si_v, sf_v, isem):
            pltpu.make_async_copy(si_hbm.at[0, pl.ds(b0, _BW)], si_v,
                                  isem).wait()
            pltpu.make_async_copy(sf_hbm.at[0, pl.ds(b0, _BW), :], sf_v,
                                  isem).wait()

        def wait_out(out_v, sc_v, osem):
            pltpu.make_async_copy(out_v, seq_hbm.at[0, pl.ds(b0, _BW), :],
                                  osem).wait()
            pltpu.make_async_copy(sc_v, score_hbm.at[0, pl.ds(b0, _BW), :],
                                  osem).wait()

        start_in(0, si0, sf0, isem0)
        start_in(1, si1, sf1, isem1)

        def body(step, carry):
            for p in range(2):
                si_v, sf_v, out_v, sc_v, isem, osem = bufs[p]
                l = 2 * step + p
                wait_in(si_v, sf_v, isem)

                @pl.when(step > 0)
                def _():
                    wait_out(out_v, sc_v, osem)

                for g in range(_NG):
                    rowv = iota + (g * _LANES)
                    siv = si_v[pl.ds(g * _LANES, _LANES)] + 1
                    for j in range(3):
                        lv = plsc.load_gather(lp_v, [siv, _splat(j)])
                        plsc.store_scatter(out_v, [rowv, _splat(12 + j)], lv)
                    vs = []
                    for j in range(5):
                        vj = plsc.load_gather(sf_v, [rowv, _splat(j)])
                        vs.append(vj)
                        plsc.store_scatter(out_v, [rowv, _splat(15 + j)], vj)
                    speed = 1.0 / (vs[2] + 1.0)
                    plsc.store_scatter(sc_v, [rowv, _splat(0)], speed)
                    plsc.store_scatter(sc_v, [rowv, _splat(1)], vs[3])
                    plsc.store_scatter(sc_v, [rowv, _splat(2)], vs[4])
                pltpu.async_copy(out_v, seq_hbm.at[l, pl.ds(b0, _BW), :],
                                 osem)
                pltpu.async_copy(sc_v, score_hbm.at[l, pl.ds(b0, _BW), :],
                                 osem)

                @pl.when(step < _L // 2 - 1)
                def _():
                    start_in(l + 2, si_v, sf_v, isem)
            return carry

        lax.fori_loop(0, _L // 2, body, 0)
        wait_out(out0, sc0, osem0)
        wait_out(out1, sc1, osem1)

    return sc


_sc_kernel = _sc_build()


def kernel(meta_int, meta_float, seq_int, seq_float, W_locS, dw_table,
           ts_table, lp_table):
    mi = meta_int.astype(jnp.int32)
    mf = jnp.squeeze(meta_float, -1)
    si = jnp.squeeze(seq_int, -1).astype(jnp.int32)
    w12 = jnp.zeros((_NPOI, 12), jnp.float32).at[:, 0:3].set(W_locS.T)
    dw12 = jnp.zeros((7, 12), jnp.float32).at[:, 3:6].set(dw_table)
    ts12 = jnp.zeros((48, 12), jnp.float32).at[:, 6:12].set(ts_table)
    s_depart = pl.pallas_call(
        _sdep_body,
        out_shape=jax.ShapeDtypeStruct((_B, 12), jnp.float32),
    )(mf, mi, w12, dw12, ts12)
    seq, score = _sc_kernel(s_depart, si, seq_float, lp_table)
    return (s_depart, seq, score)


# speedup vs baseline: 21.5048x; 8.7043x over previous
import functools

import jax
import jax.numpy as jnp
from jax import lax
from jax.experimental import pallas as pl
from jax.experimental.pallas import tpu as pltpu
from jax.experimental.pallas import tpu_sc as plsc

_B = 4096
_L = 200
_NPOI = 100
_NW = 32
_LANES = 16

_CR = 4
_NCK = _L // _CR
_NQ = 11 * _NCK
_AR = 8
_NA = 12 * (_L // _AR)


def _sdep_t_body(mft_ref, mit_ref, w12t_ref, dw12t_ref, ts12t_ref, out_ref):
    mft = mft_ref[...]
    mit = mit_ref[...]
    hi = lax.Precision.HIGHEST
    locs = jnp.dot(w12t_ref[...], mft, preferred_element_type=jnp.float32,
                   precision=hi)
    dwoh = (lax.broadcasted_iota(jnp.int32, (7, _B), 0) == mit[0:1, :]
            ).astype(jnp.float32)
    tsoh = (lax.broadcasted_iota(jnp.int32, (48, _B), 0) == mit[1:2, :]
            ).astype(jnp.float32)
    out_ref[...] = (locs
                    + jnp.dot(dw12t_ref[...], dwoh,
                              preferred_element_type=jnp.float32, precision=hi)
                    + jnp.dot(ts12t_ref[...], tsoh,
                              preferred_element_type=jnp.float32, precision=hi))


def _splat(v):
    return jnp.full((_LANES,), v, jnp.int32)


def _sc_build():
    mesh = plsc.VectorSubcoreMesh(core_axis_name="c", subcore_axis_name="s")
    scratch = [
        pltpu.VMEM((3, _NPOI + 1), jnp.float32),
        pltpu.VMEM((_CR, _B), jnp.float32),
        pltpu.VMEM((_CR, _B), jnp.float32),
        pltpu.VMEM((_CR, _B), jnp.float32),
        pltpu.VMEM((_CR, _B), jnp.int32),
        pltpu.VMEM((_CR, _B), jnp.int32),
        pltpu.VMEM((_CR, _B), jnp.int32),
        pltpu.VMEM_SHARED((12, _AR, _B), jnp.float32),
        pltpu.SemaphoreType.DMA,
        pltpu.SemaphoreType.DMA,
        pltpu.SemaphoreType.DMA,
        pltpu.SemaphoreType.DMA,
        pltpu.SemaphoreType.DMA,
        pltpu.SemaphoreType.DMA,
        pltpu.SemaphoreType.DMA,
    ]

    @functools.partial(
        pl.kernel, mesh=mesh,
        out_type=(jax.ShapeDtypeStruct((20, _L, _B), jnp.float32),
                  jax.ShapeDtypeStruct((3, _L, _B), jnp.float32)),
        scratch_types=scratch,
        compiler_params=pltpu.CompilerParams(needs_layout_passes=False))
    def sc(sdepT_hbm, si_hbm, sft_hbm, lpt_hbm, seqT_hbm, scoreT_hbm,
           lpt_v, cbuf0, cbuf1, cbuf2, sibuf0, sibuf1, sibuf2, sdep_s,
           isem0, isem1, isem2, osem0, osem1, osem2, asem):
        cid = lax.axis_index("c")
        sid = lax.axis_index("s")
        wid = sid * 2 + cid
        cbufs = (cbuf0, cbuf1, cbuf2)
        sibufs = (sibuf0, sibuf1, sibuf2)
        isems = (isem0, isem1, isem2)
        osems = (osem0, osem1, osem2)
        iota = lax.iota(jnp.int32, _LANES)

        pltpu.sync_copy(lpt_hbm, lpt_v)

        @pl.when(sid < 12)
        def _():
            for r in range(_AR):
                pltpu.async_copy(sdepT_hbm.at[sid, :], sdep_s.at[sid, r, :],
                                 asem)
            for r in range(_AR):
                pltpu.make_async_copy(sdepT_hbm.at[0, :],
                                      sdep_s.at[0, 0, :], asem).wait()

        plsc.subcore_barrier()

        n_a = (_NA - wid + _NW - 1) // _NW

        def a_body(i, carry):
            q = wid + _NW * i
            j = q // (_L // _AR)
            l0 = (q % (_L // _AR)) * _AR

            @pl.when(i >= 4)
            def _():
                pltpu.make_async_copy(sdep_s.at[0],
                                      seqT_hbm.at[0, pl.ds(0, _AR), :],
                                      asem).wait()

            pltpu.async_copy(sdep_s.at[j], seqT_hbm.at[j, pl.ds(l0, _AR), :],
                             asem)
            return carry

        lax.fori_loop(0, n_a, a_body, 0)

        def a_drain(i, carry):
            pltpu.make_async_copy(sdep_s.at[0],
                                  seqT_hbm.at[0, pl.ds(0, _AR), :],
                                  asem).wait()
            return carry

        lax.fori_loop(0, jnp.minimum(n_a, 4), a_drain, 0)

        n = (_NQ - wid + _NW - 1) // _NW

        def params(q):
            c50 = q // _NCK
            l0 = (q % _NCK) * _CR
            is_gather = c50 >= 8
            is_recip = c50 == 7
            to_score = (c50 >= 5) & (c50 < 8)
            srcp = jnp.where(c50 < 5, c50, jnp.where(c50 < 7, c50 - 2, 2))
            dst_seq = jnp.where(c50 < 5, 15 + c50, 12 + (c50 - 8))
            dst_sco = jnp.where(c50 == 7, 0, c50 - 4)
            lp_row = jnp.maximum(c50 - 8, 0)
            return (l0, is_gather, is_recip, to_score, srcp, dst_seq,
                    dst_sco, lp_row)

        def start_in(q, p):
            (l0, is_gather, _, _, srcp, _, _, _) = params(q)

            @pl.when(jnp.logical_not(is_gather))
            def _():
                pltpu.async_copy(sft_hbm.at[srcp, pl.ds(l0, _CR), :],
                                 cbufs[p], isems[p])

            @pl.when(is_gather)
            def _():
                pltpu.async_copy(si_hbm.at[pl.ds(l0, _CR), :],
                                 sibufs[p], isems[p])

        def wait_in(p):
            pltpu.make_async_copy(sft_hbm.at[0, pl.ds(0, _CR), :],
                                  cbufs[p], isems[p]).wait()

        def wait_out(p):
            pltpu.make_async_copy(cbufs[p],
                                  seqT_hbm.at[0, pl.ds(0, _CR), :],
                                  osems[p]).wait()

        start_in(wid, 0)

        def step(i, p):
            q = wid + _NW * i
            (l0, is_gather, is_recip, to_score, _, dst_seq, dst_sco,
             lp_row) = params(q)
            p1 = (p + 1) % 3

            @pl.when(i + 1 < n)
            def _():
                @pl.when(i >= 2)
                def _():
                    wait_out(p1)
                start_in(q + _NW, p1)

            wait_in(p)

            @pl.when(is_recip)
            def _():
                def rec_body(k, carry):
                    for r in range(_CR):
                        for u in range(4):
                            sl = pl.ds((k * 4 + u) * _LANES, _LANES)
                            v = cbufs[p][r, sl]
                            cbufs[p][r, sl] = 1.0 / (v + 1.0)
                    return carry
                lax.fori_loop(0, _B // (4 * _LANES), rec_body, 0)

            @pl.when(is_gather)
            def _():
                rowv = jnp.full((_LANES,), lp_row, jnp.int32)

                def g_body(k, carry):
                    for r in range(_CR):
                        for u in range(4):
                            sl = pl.ds((k * 4 + u) * _LANES, _LANES)
                            idx = sibufs[p][r, sl] + 1
                            cbufs[p][r, sl] = plsc.load_gather(
                                lpt_v, [rowv, idx])
                    return carry
                lax.fori_loop(0, _B // (4 * _LANES), g_body, 0)

            @pl.when(to_score)
            def _():
                pltpu.async_copy(cbufs[p],
                                 scoreT_hbm.at[dst_sco, pl.ds(l0, _CR), :],
                                 osems[p])

            @pl.when(jnp.logical_not(to_score))
            def _():
                pltpu.async_copy(cbufs[p],
                                 seqT_hbm.at[dst_seq, pl.ds(l0, _CR), :],
                                 osems[p])

        def outer(t, carry):
            for p in range(3):
                i = 3 * t + p

                @pl.when(i < n)
                def _():
                    step(i, p)
            return carry

        lax.fori_loop(0, (n + 2) // 3, outer, 0)
        for p in range(3):
            wait_out(p)

    return sc


_sc_kernel = _sc_build()


def kernel(meta_int, meta_float, seq_int, seq_float, W_locS, dw_table,
           ts_table, lp_table):
    mi_t = jnp.transpose(meta_int.astype(jnp.int32))
    mf_t = jnp.transpose(jnp.squeeze(meta_float, -1))
    si_p = jnp.squeeze(seq_int, -1).astype(jnp.int32)
    sf_t = jnp.transpose(seq_float, (2, 0, 1))
    lp_t = jnp.transpose(lp_table)
    w12t = jnp.zeros((12, _NPOI), jnp.float32).at[0:3].set(W_locS)
    dw12t = jnp.zeros((12, 7), jnp.float32).at[3:6].set(dw_table.T)
    ts12t = jnp.zeros((12, 48), jnp.float32).at[6:12].set(ts_table.T)
    sdep_t = pl.pallas_call(
        _sdep_t_body,
        out_shape=jax.ShapeDtypeStruct((12, _B), jnp.float32),
    )(mf_t, mi_t, w12t, dw12t, ts12t)
    seq_t, score_t = _sc_kernel(sdep_t, si_p, sf_t, lp_t)
    return (jnp.transpose(sdep_t),
            jnp.transpose(seq_t, (1, 2, 0)),
            jnp.transpose(score_t, (1, 2, 0)))


# speedup vs baseline: 22.9698x; 1.0681x over previous
import functools

import jax
import jax.numpy as jnp
from jax import lax
from jax.experimental import pallas as pl
from jax.experimental.pallas import tpu as pltpu
from jax.experimental.pallas import tpu_sc as plsc

_B = 4096
_L = 200
_NPOI = 100
_NW = 32
_LANES = 16

_CR = 2
_NCK = _L // _CR
_NQ = 8 * _NCK
_NS = 6
_PD = 3
_AR = 8
_NA = 12 * (_L // _AR)


def _sdep_t_body(mft_ref, mit_ref, w12t_ref, dw12t_ref, ts12t_ref, out_ref):
    mft = mft_ref[...]
    mit = mit_ref[...]
    hi = lax.Precision.HIGHEST
    locs = jnp.dot(w12t_ref[...], mft, preferred_element_type=jnp.float32,
                   precision=hi)
    dwoh = (lax.broadcasted_iota(jnp.int32, (7, _B), 0) == mit[0:1, :]
            ).astype(jnp.float32)
    tsoh = (lax.broadcasted_iota(jnp.int32, (48, _B), 0) == mit[1:2, :]
            ).astype(jnp.float32)
    out_ref[...] = (locs
                    + jnp.dot(dw12t_ref[...], dwoh,
                              preferred_element_type=jnp.float32, precision=hi)
                    + jnp.dot(ts12t_ref[...], tsoh,
                              preferred_element_type=jnp.float32, precision=hi))


def _score_body(s2_ref, s3_ref, s4_ref, out_ref):
    out_ref[0] = 1.0 / (s2_ref[0] + 1.0)
    out_ref[1] = s3_ref[0]
    out_ref[2] = s4_ref[0]


_SCORE_ROWS = 40


def _score_kernel(sf_t):
    grid = _L // _SCORE_ROWS
    blk = (1, _SCORE_ROWS, _B)
    return pl.pallas_call(
        _score_body,
        grid=(grid,),
        in_specs=[
            pl.BlockSpec(blk, lambda g: (2, g, 0)),
            pl.BlockSpec(blk, lambda g: (3, g, 0)),
            pl.BlockSpec(blk, lambda g: (4, g, 0)),
        ],
        out_specs=pl.BlockSpec((3, _SCORE_ROWS, _B), lambda g: (0, g, 0)),
        out_shape=jax.ShapeDtypeStruct((3, _L, _B), jnp.float32),
    )(sf_t, sf_t, sf_t)


def _sc_build():
    mesh = plsc.VectorSubcoreMesh(core_axis_name="c", subcore_axis_name="s")
    scratch = (
        [pltpu.VMEM((3, _NPOI + 1), jnp.float32)]
        + [pltpu.VMEM((_CR, _B), jnp.float32) for _ in range(_NS)]
        + [pltpu.VMEM((_CR * _B,), jnp.int32) for _ in range(_NS)]
        + [pltpu.VMEM_SHARED((12, _AR, _B), jnp.float32)]
        + [pltpu.SemaphoreType.DMA for _ in range(2 * _NS)]
        + [pltpu.SemaphoreType.DMA]
    )

    @functools.partial(
        pl.kernel, mesh=mesh,
        out_type=jax.ShapeDtypeStruct((20, _L, _B), jnp.float32),
        scratch_types=scratch,
        compiler_params=pltpu.CompilerParams(needs_layout_passes=False))
    def sc(sdepT_hbm, si_hbm, sft_hbm, lpt_hbm, seqT_hbm, *scr):
        lpt_v = scr[0]
        cbufs = scr[1:1 + _NS]
        sibufs = scr[1 + _NS:1 + 2 * _NS]
        sdep_s = scr[1 + 2 * _NS]
        isems = scr[2 + 2 * _NS:2 + 2 * _NS + _NS]
        osems = scr[2 + 2 * _NS + _NS:2 + 2 * _NS + 2 * _NS]
        asem = scr[-1]
        cid = lax.axis_index("c")
        sid = lax.axis_index("s")
        wid = sid * 2 + cid
        iota = lax.iota(jnp.int32, _LANES)

        pltpu.sync_copy(lpt_hbm, lpt_v)

        @pl.when(sid < 12)
        def _():
            for r in range(_AR):
                pltpu.async_copy(sdepT_hbm.at[sid, :], sdep_s.at[sid, r, :],
                                 asem)
            for r in range(_AR):
                pltpu.make_async_copy(sdepT_hbm.at[0, :],
                                      sdep_s.at[0, 0, :], asem).wait()

        plsc.subcore_barrier()

        n = (_NQ - wid + _NW - 1) // _NW

        def params(q):
            cp = q // _NCK
            l0 = (q % _NCK) * _CR
            is_gather = cp >= 5
            srcp = jnp.where(cp < 5, cp, 0)
            dstp = jnp.where(cp < 5, 15 + cp, 12 + (cp - 5))
            lp_row = jnp.maximum(cp - 5, 0)
            return l0, is_gather, srcp, dstp, lp_row

        def start_in(q, p):
            l0, is_gather, srcp, _, _ = params(q)

            @pl.when(jnp.logical_not(is_gather))
            def _():
                pltpu.async_copy(sft_hbm.at[srcp, pl.ds(l0, _CR), :],
                                 cbufs[p], isems[p])

            @pl.when(is_gather)
            def _():
                pltpu.async_copy(si_hbm.at[pl.ds(l0 * _B, _CR * _B)],
                                 sibufs[p], isems[p])

        def wait_in(p):
            pltpu.make_async_copy(sft_hbm.at[0, pl.ds(0, _CR), :],
                                  cbufs[p], isems[p]).wait()

        def wait_out(p):
            pltpu.make_async_copy(cbufs[p],
                                  seqT_hbm.at[0, pl.ds(0, _CR), :],
                                  osems[p]).wait()

        for p in range(_PD):
            @pl.when(p < n)
            def _(p=p):
                start_in(wid + _NW * p, p)

        n_a = (_NA - wid + _NW - 1) // _NW

        def a_body(i, carry):
            q = wid + _NW * i
            j = q // (_L // _AR)
            l0 = (q % (_L // _AR)) * _AR

            @pl.when(i >= 4)
            def _():
                pltpu.make_async_copy(sdep_s.at[0],
                                      seqT_hbm.at[0, pl.ds(0, _AR), :],
                                      asem).wait()

            pltpu.async_copy(sdep_s.at[j], seqT_hbm.at[j, pl.ds(l0, _AR), :],
                             asem)
            return carry

        lax.fori_loop(0, n_a, a_body, 0)

        def step(i, p):
            q = wid + _NW * i
            l0, is_gather, _, dstp, lp_row = params(q)
            pn = (p + _PD) % _NS

            @pl.when(i + _PD < n)
            def _():
                @pl.when(i >= _NS - _PD)
                def _():
                    wait_out(pn)
                start_in(q + _NW * _PD, pn)

            wait_in(p)

            @pl.when(is_gather)
            def _():
                rowv = jnp.full((_LANES,), lp_row, jnp.int32)

                def g_body(k, carry):
                    for r in range(_CR):
                        for u in range(4):
                            c = (k * 4 + u) * _LANES
                            idx = sibufs[p][pl.ds(r * _B + c, _LANES)] + 1
                            cbufs[p][r, pl.ds(c, _LANES)] = plsc.load_gather(
                                lpt_v, [rowv, idx])
                    return carry
                lax.fori_loop(0, _B // (4 * _LANES), g_body, 0)

            pltpu.async_copy(cbufs[p], seqT_hbm.at[dstp, pl.ds(l0, _CR), :],
                             osems[p])

        def outer(t, carry):
            for p in range(_NS):
                i = _NS * t + p

                @pl.when(i < n)
                def _(i=i, p=p):
                    step(i, p)
            return carry

        lax.fori_loop(0, (n + _NS - 1) // _NS, outer, 0)
        for p in range(_NS):
            wait_out(p)

        def a_drain(i, carry):
            pltpu.make_async_copy(sdep_s.at[0],
                                  seqT_hbm.at[0, pl.ds(0, _AR), :],
                                  asem).wait()
            return carry

        lax.fori_loop(0, jnp.minimum(n_a, 4), a_drain, 0)

    return sc


_sc_kernel = _sc_build()


def kernel(meta_int, meta_float, seq_int, seq_float, W_locS, dw_table,
           ts_table, lp_table):
    mi_t = jnp.transpose(meta_int.astype(jnp.int32))
    mf_t = jnp.transpose(jnp.squeeze(meta_float, -1))
    si_f = jnp.reshape(seq_int.astype(jnp.int32), (_L * _B,))
    sf_t = jnp.transpose(seq_float, (2, 0, 1))
    lp_t = jnp.transpose(lp_table)
    w12t = jnp.zeros((12, _NPOI), jnp.float32).at[0:3].set(W_locS)
    dw12t = jnp.zeros((12, 7), jnp.float32).at[3:6].set(dw_table.T)
    ts12t = jnp.zeros((12, 48), jnp.float32).at[6:12].set(ts_table.T)
    sdep_t = pl.pallas_call(
        _sdep_t_body,
        out_shape=jax.ShapeDtypeStruct((12, _B), jnp.float32),
    )(mf_t, mi_t, w12t, dw12t, ts12t)
    score_t = _score_kernel(sf_t)
    seq_t = _sc_kernel(sdep_t, si_f, sf_t, lp_t)
    return (jnp.transpose(sdep_t),
            jnp.transpose(seq_t, (1, 2, 0)),
            jnp.transpose(score_t, (1, 2, 0)))


# speedup vs baseline: 25.3519x; 1.1037x over previous
import functools

import jax
import jax.numpy as jnp
from jax import lax
from jax.experimental import pallas as pl
from jax.experimental.pallas import tpu as pltpu
from jax.experimental.pallas import tpu_sc as plsc

_B = 4096
_L = 200
_NPOI = 100
_NW = 32
_LANES = 16

_KR = 8
_NC = 5 * (_L // _KR)
_GR = 2
_NG = 3 * (_L // _GR)
_AR = 8
_NA = 12 * (_L // _AR)


def _sdep_t_body(mft_ref, mit_ref, w12t_ref, dw12t_ref, ts12t_ref, out_ref):
    mft = mft_ref[...]
    mit = mit_ref[...]
    hi = lax.Precision.HIGHEST
    locs = jnp.dot(w12t_ref[...], mft, preferred_element_type=jnp.float32,
                   precision=hi)
    dwoh = (lax.broadcasted_iota(jnp.int32, (7, _B), 0) == mit[0:1, :]
            ).astype(jnp.float32)
    tsoh = (lax.broadcasted_iota(jnp.int32, (48, _B), 0) == mit[1:2, :]
            ).astype(jnp.float32)
    out_ref[...] = (locs
                    + jnp.dot(dw12t_ref[...], dwoh,
                              preferred_element_type=jnp.float32, precision=hi)
                    + jnp.dot(ts12t_ref[...], tsoh,
                              preferred_element_type=jnp.float32, precision=hi))


def _score_body(s2_ref, s3_ref, s4_ref, out_ref):
    out_ref[0] = 1.0 / (s2_ref[0] + 1.0)
    out_ref[1] = s3_ref[0]
    out_ref[2] = s4_ref[0]


_SCORE_ROWS = 40


def _score_kernel(sf_t):
    grid = _L // _SCORE_ROWS
    blk = (1, _SCORE_ROWS, _B)
    return pl.pallas_call(
        _score_body,
        grid=(grid,),
        in_specs=[
            pl.BlockSpec(blk, lambda g: (2, g, 0)),
            pl.BlockSpec(blk, lambda g: (3, g, 0)),
            pl.BlockSpec(blk, lambda g: (4, g, 0)),
        ],
        out_specs=pl.BlockSpec((3, _SCORE_ROWS, _B), lambda g: (0, g, 0)),
        out_shape=jax.ShapeDtypeStruct((3, _L, _B), jnp.float32),
    )(sf_t, sf_t, sf_t)


def _sc_build():
    mesh = plsc.VectorSubcoreMesh(core_axis_name="c", subcore_axis_name="s")
    scratch = (
        [pltpu.VMEM((3, _NPOI + 1), jnp.float32)]
        + [pltpu.VMEM((_KR, _B), jnp.float32) for _ in range(2)]
        + [pltpu.VMEM((_GR * _B,), jnp.int32) for _ in range(2)]
        + [pltpu.VMEM((_GR, _B), jnp.float32) for _ in range(2)]
        + [pltpu.VMEM_SHARED((12, _AR, _B), jnp.float32)]
        + [pltpu.SemaphoreType.DMA for _ in range(9)]
    )

    @functools.partial(
        pl.kernel, mesh=mesh,
        out_type=jax.ShapeDtypeStruct((20, _L, _B), jnp.float32),
        scratch_types=scratch,
        compiler_params=pltpu.CompilerParams(needs_layout_passes=False))
    def sc(sdepT_hbm, si_hbm, sft_hbm, lpt_hbm, seqT_hbm,
           lpt_v, cbuf0, cbuf1, sibuf0, sibuf1, gout0, gout1, sdep_s,
           csem0, csem1, cosem0, cosem1, gsem0, gsem1, gosem0, gosem1,
           asem):
        cbufs = (cbuf0, cbuf1)
        sibufs = (sibuf0, sibuf1)
        gouts = (gout0, gout1)
        csems = (csem0, csem1)
        cosems = (cosem0, cosem1)
        gsems = (gsem0, gsem1)
        gosems = (gosem0, gosem1)
        cid = lax.axis_index("c")
        sid = lax.axis_index("s")
        wid = sid * 2 + cid

        pltpu.sync_copy(lpt_hbm, lpt_v)

        n_c = (_NC - wid + _NW - 1) // _NW
        n_g = (_NG - wid + _NW - 1) // _NW
        n_a = (_NA - wid + _NW - 1) // _NW

        def cpar(q):
            return q // (_L // _KR), 15 + q // (_L // _KR), (q % (_L // _KR)) * _KR

        def gpar(q):
            return q // (_L // _GR), 12 + q // (_L // _GR), (q % (_L // _GR)) * _GR

        def c_in(q, p):
            srcp, _, l0 = cpar(q)
            pltpu.async_copy(sft_hbm.at[srcp, pl.ds(l0, _KR), :], cbufs[p],
                             csems[p])

        def g_in(q, p):
            _, _, l0 = gpar(q)
            pltpu.async_copy(si_hbm.at[pl.ds(l0 * _B, _GR * _B)], sibufs[p],
                             gsems[p])

        def c_wait_out(p):
            pltpu.make_async_copy(cbufs[p], seqT_hbm.at[0, pl.ds(0, _KR), :],
                                  cosems[p]).wait()

        def g_wait_out(p):
            pltpu.make_async_copy(gouts[p], seqT_hbm.at[0, pl.ds(0, _GR), :],
                                  gosems[p]).wait()

        for p in range(2):
            @pl.when(p < n_g)
            def _(p=p):
                g_in(wid + _NW * p, p)

            @pl.when(p < n_c)
            def _(p=p):
                c_in(wid + _NW * p, p)

        @pl.when(sid < 12)
        def _():
            for r in range(_AR):
                pltpu.async_copy(sdepT_hbm.at[sid, :], sdep_s.at[sid, r, :],
                                 asem)
            for r in range(_AR):
                pltpu.make_async_copy(sdepT_hbm.at[0, :],
                                      sdep_s.at[0, 0, :], asem).wait()

        plsc.subcore_barrier()

        def a_body(i, carry):
            q = wid + _NW * i
            j = q // (_L // _AR)
            l0 = (q % (_L // _AR)) * _AR

            @pl.when(i >= 6)
            def _():
                pltpu.make_async_copy(sdep_s.at[0],
                                      seqT_hbm.at[0, pl.ds(0, _AR), :],
                                      asem).wait()

            pltpu.async_copy(sdep_s.at[j], seqT_hbm.at[j, pl.ds(l0, _AR), :],
                             asem)
            return carry

        lax.fori_loop(0, n_a, a_body, 0)

        def g_step(i, p):
            q = wid + _NW * i
            lp_row, dstp, l0 = gpar(q)
            pltpu.make_async_copy(si_hbm.at[pl.ds(0, _GR * _B)], sibufs[p],
                                  gsems[p]).wait()

            @pl.when(i >= 2)
            def _():
                g_wait_out(p)

            rowv = jnp.full((_LANES,), lp_row, jnp.int32)

            def g_body(k, carry):
                for r in range(_GR):
                    for u in range(4):
                        c = (k * 4 + u) * _LANES
                        idx = sibufs[p][pl.ds(r * _B + c, _LANES)] + 1
                        gouts[p][r, pl.ds(c, _LANES)] = plsc.load_gather(
                            lpt_v, [rowv, idx])
                return carry

            lax.fori_loop(0, _B // (4 * _LANES), g_body, 0)
            pltpu.async_copy(gouts[p], seqT_hbm.at[dstp, pl.ds(l0, _GR), :],
                             gosems[p])

            @pl.when(i + 2 < n_g)
            def _():
                g_in(q + 2 * _NW, p)

        def g_outer(t, carry):
            for p in range(2):
                i = 2 * t + p

                @pl.when(i < n_g)
                def _(i=i, p=p):
                    g_step(i, p)
            return carry

        lax.fori_loop(0, (n_g + 1) // 2, g_outer, 0)

        def c_step(i, p):
            q = wid + _NW * i
            _, dstp, l0 = cpar(q)
            pltpu.make_async_copy(sft_hbm.at[0, pl.ds(0, _KR), :], cbufs[p],
                                  csems[p]).wait()
            pltpu.async_copy(cbufs[p], seqT_hbm.at[dstp, pl.ds(l0, _KR), :],
                             cosems[p])

            @pl.when(i + 2 < n_c)
            def _():
                c_wait_out(p)
                c_in(q + 2 * _NW, p)

        def c_outer(t, carry):
            for p in range(2):
                i = 2 * t + p

                @pl.when(i < n_c)
                def _(i=i, p=p):
                    c_step(i, p)
            return carry

        lax.fori_loop(0, (n_c + 1) // 2, c_outer, 0)

        for p in range(2):
            @pl.when(n_c >= p + 1)
            def _(p=p):
                c_wait_out(p)

            @pl.when(n_g >= p + 1)
            def _(p=p):
                g_wait_out(p)

        def a_drain(i, carry):
            pltpu.make_async_copy(sdep_s.at[0],
                                  seqT_hbm.at[0, pl.ds(0, _AR), :],
                                  asem).wait()
            return carry

        lax.fori_loop(0, jnp.minimum(n_a, 6), a_drain, 0)

    return sc


_sc_kernel = _sc_build()


def kernel(meta_int, meta_float, seq_int, seq_float, W_locS, dw_table,
           ts_table, lp_table):
    mi_t = jnp.transpose(meta_int.astype(jnp.int32))
    mf_t = jnp.transpose(jnp.squeeze(meta_float, -1))
    si_f = jnp.reshape(seq_int.astype(jnp.int32), (_L * _B,))
    sf_t = jnp.transpose(seq_float, (2, 0, 1))
    lp_t = jnp.transpose(lp_table)
    w12t = jnp.zeros((12, _NPOI), jnp.float32).at[0:3].set(W_locS)
    dw12t = jnp.zeros((12, 7), jnp.float32).at[3:6].set(dw_table.T)
    ts12t = jnp.zeros((12, 48), jnp.float32).at[6:12].set(ts_table.T)
    sdep_t = pl.pallas_call(
        _sdep_t_body,
        out_shape=jax.ShapeDtypeStruct((12, _B), jnp.float32),
    )(mf_t, mi_t, w12t, dw12t, ts12t)
    score_t = _score_kernel(sf_t)
    seq_t = _sc_kernel(sdep_t, si_f, sf_t, lp_t)
    return (jnp.transpose(sdep_t),
            jnp.transpose(seq_t, (1, 2, 0)),
            jnp.transpose(score_t, (1, 2, 0)))
